# baseline (speedup 1.0000x reference)
import functools

import numpy as np
import jax
import jax.numpy as jnp
from jax import lax
from jax.experimental import pallas as pl
from jax.experimental.pallas import tpu as pltpu
from jax.experimental.pallas import tpu_sc as plsc

_NS = 32
_SH = 9
_WN = _NS * _SH
_NCLS = 14
_NPROT = 50000
_NPHARM = 10000
_NTAB = _NPROT + _NPHARM
_E1 = 320000
_E2 = 128000
_NE_ALL = 2 * _E1 + 2 * _E2

_EB = 2560
_NW = 32
_GC = 2000
_SCC = 80

_R_NP = np.zeros((_NS, _WN), np.float32)
for _i in range(_NS):
    _R_NP[_i, _i * _SH:(_i + 1) * _SH] = 1.0
_S_NP = np.zeros((_SH, _WN), np.float32)
for _j in range(_SH):
    _S_NP[_j, _j::_SH] = 1.0

_S3, _S5, _S15 = 3.0 ** 0.5, 5.0 ** 0.5, 15.0 ** 0.5
_C_NP = np.zeros((9, 9), np.float32)
_C_NP[1, 1] = _S3
_C_NP[2, 2] = _S3
_C_NP[0, 3] = _S3
_C_NP[6, 4] = _S15
_C_NP[7, 5] = _S15
_C_NP[5, 6] = 1.5 * _S5
_C_NP[8, 7] = _S15
_C_NP[3, 8] = 0.5 * _S15
_C_NP[4, 8] = -0.5 * _S15
_E_NP = np.zeros((9,), np.float32)
_E_NP[0] = 1.0
_E_NP[6] = -0.5 * _S5
_CS_NP = _C_NP @ _S_NP
_ES_NP = (_E_NP @ _S_NP).reshape(1, _WN)


def _pharm_body(x_ref, w1_ref, b1_ref, w2_ref, b2_ref, o_ref):
    h = jnp.maximum(x_ref[...] @ w1_ref[...] + b1_ref[...], 0.0)
    o_ref[...] = h @ w2_ref[...] + b2_ref[...]


def _pharm_call(x, w1, b1, w2, b2):
    return pl.pallas_call(
        _pharm_body,
        out_shape=jax.ShapeDtypeStruct((_NPHARM, _NS), jnp.float32),
    )(x, w1, b1, w2, b2)


def _prot_body(idx_ref, tab_ref, w_ref, b_ref, o_ref):
    t2 = jnp.maximum(tab_ref[...], 0.0) @ w_ref[...] + b_ref[...]
    idx = idx_ref[...]
    onehot = (idx == lax.broadcasted_iota(jnp.int32, (idx.shape[0], _NCLS), 1)
              ).astype(jnp.float32)
    o_ref[...] = onehot @ t2


def _prot_call(idx2d, tab, w, b):
    blk = 10000
    return pl.pallas_call(
        _prot_body,
        grid=(_NPROT // blk,),
        in_specs=[
            pl.BlockSpec((blk, 1), lambda i: (i, 0)),
            pl.BlockSpec((_NCLS, _NS), lambda i: (0, 0)),
            pl.BlockSpec((_NS, _NS), lambda i: (0, 0)),
            pl.BlockSpec((1, _NS), lambda i: (0, 0)),
        ],
        out_specs=pl.BlockSpec((blk, _NS), lambda i: (i, 0)),
        out_shape=jax.ShapeDtypeStruct((_NPROT, _NS), jnp.float32),
    )(idx2d, tab, w, b)


@functools.lru_cache(maxsize=None)
def _sc_mesh():
    return plsc.VectorSubcoreMesh(core_axis_name="c", subcore_axis_name="s")


@functools.lru_cache(maxsize=None)
def _sc_gather_kernel():
    @functools.partial(
        pl.kernel,
        out_type=jax.ShapeDtypeStruct((_NE_ALL, _NS), jnp.bfloat16),
        mesh=_sc_mesh(),
        scratch_types=[
            pltpu.VMEM((_GC,), jnp.int32),
            pltpu.VMEM((_GC, _NS), jnp.bfloat16),
            pltpu.SemaphoreType.DMA,
        ],
        compiler_params=pltpu.CompilerParams(use_tc_tiling_on_sc=False),
    )
    def _sc_gather(prot_hbm, pharm_hbm, i1_hbm, i2_hbm, i3_hbm, i4_hbm,
                   out_hbm, idx_v, rows_v, sem):
        wid = lax.axis_index("s") * 2 + lax.axis_index("c")

        def seg(tab_hbm, idx_hbm, n, out_base):
            per = n // _NW
            base = wid * per

            def body(j, carry):
                off = base + j * _GC
                pltpu.sync_copy(idx_hbm.at[pl.ds(off, _GC)], idx_v)
                pltpu.async_copy(tab_hbm.at[idx_v], rows_v, sem).wait()
                pltpu.sync_copy(rows_v, out_hbm.at[pl.ds(out_base + off, _GC)])
                return carry

            lax.fori_loop(0, per // _GC, body, 0)

        seg(prot_hbm, i1_hbm, _E1, 0)
        seg(pharm_hbm, i2_hbm, _E1, _E1)
        seg(pharm_hbm, i3_hbm, _E2, 2 * _E1)
        seg(pharm_hbm, i4_hbm, _E2, 2 * _E1 + _E2)

    return _sc_gather


def _edge_body(vt_ref, xs_ref, xd_ref, dw1_ref, db1_ref, dw2_ref, db2_ref,
               cw1_ref, cb1_ref, cw2_ref, cb2_ref, r_ref, cs_ref, es_ref,
               off_ref, p_ref, o_ref, *, B):
    bf = jnp.bfloat16
    dot = functools.partial(jax.lax.dot, preferred_element_type=jnp.float32)
    dot0 = lambda a, b: lax.dot_general(
        a, b, (((0,), (0,)), ((), ())), preferred_element_type=jnp.float32)

    vt = vt_ref[...]
    ss = jnp.sum(vt * vt, axis=0, keepdims=True)
    rinv = lax.rsqrt(ss + 1e-12)
    dist = ss * rinv
    u = vt * rinv
    ur = jnp.concatenate([u[1:3], u[0:1]], axis=0)
    mono = jnp.concatenate([u, u * u, u * ur], axis=0)
    sh_tile = dot0(mono.astype(bf), cs_ref[...].astype(bf)) + es_ref[...]

    step = 6.0 / 49.0
    dd = dist - off_ref[...]
    g = jnp.exp((-0.5 / (step * step)) * dd * dd)
    hd = jnp.maximum(dot0(dw1_ref[...], g) + db1_ref[...], 0.0)

    cw1 = cw1_ref[...]
    dcw = dot(dw2_ref[...], cw1[0:_NS])
    bfold = cb1_ref[...] + dot(db2_ref[...], cw1[0:_NS])
    xs = xs_ref[...].astype(bf)
    hid = jnp.maximum(
        dot0(hd.astype(bf), dcw.astype(bf))
        + dot(xs, cw1[_NS:2 * _NS].astype(bf))
        + dot(xd_ref[...].astype(bf), cw1[2 * _NS:].astype(bf))
        + bfold, 0.0)
    w = dot(hid.astype(bf), cw2_ref[...].astype(bf)) + cb2_ref[...]
    m = dot(xs, r_ref[...].astype(bf)) * sh_tile * w
    o_ref[...] = dot(m.astype(bf), p_ref[...].astype(bf))


def _edge_conv(gath, vec_t, xs_off, xd_off, n_edges,
               dw1, db1, dw2, db2, cw1, cb1, cw2, cb2, rmat, csmat, esmat,
               offmat, p32):
    B = _EB
    nb = n_edges // B
    full = lambda shape: pl.BlockSpec(shape, lambda i: (0, 0))
    return pl.pallas_call(
        functools.partial(_edge_body, B=B),
        grid=(nb,),
        in_specs=[
            pl.BlockSpec((3, B), lambda i: (0, i)),
            pl.BlockSpec((B, _NS), lambda i, o=xs_off: (i + o, 0)),
            pl.BlockSpec((B, _NS), lambda i, o=xd_off: (i + o, 0)),
            full((50, _NS)), full((_NS, 1)), full((_NS, _NS)), full((1, _NS)),
            full((3 * _NS, 3 * _NS)), full((1, 3 * _NS)),
            full((3 * _NS, _WN)), full((1, _WN)),
            full((_NS, _WN)), full((_SH, _WN)), full((1, _WN)),
            full((50, B)),
            full((_WN, _NS)),
        ],
        out_specs=pl.BlockSpec((B, _NS), lambda i: (i, 0)),
        out_shape=jax.ShapeDtypeStruct((n_edges, _NS), jnp.float32),
    )(vec_t, gath, gath, dw1, db1, dw2, db2, cw1, cb1, cw2, cb2,
      rmat, csmat, esmat, offmat, p32)


_SCT = 1000


@functools.lru_cache(maxsize=None)
def _sc_scatter_kernel(n_edges):
    per = n_edges // _NW
    nb = per // _SCT

    @functools.partial(
        pl.kernel,
        out_type=jax.ShapeDtypeStruct((2, _NPHARM, _NS), jnp.float32),
        mesh=_sc_mesh(),
        scratch_types=[
            pltpu.VMEM_SHARED((_NPHARM, _NS), jnp.float32),
            pltpu.VMEM((nb, _SCT), jnp.int32),
            pltpu.VMEM((2, _SCT, _NS), jnp.float32),
            pltpu.SemaphoreType.DMA,
            pltpu.SemaphoreType.DMA,
            pltpu.SemaphoreType.DMA,
            pltpu.SemaphoreType.DMA,
        ],
        compiler_params=pltpu.CompilerParams(use_tc_tiling_on_sc=False),
    )
    def _sc_scatter(y_hbm, d_hbm, zeros_hbm, out_hbm, acc, idx_v, rows_v,
                    si0, si1, sr0, sr1):
        c = lax.axis_index("c")
        s = lax.axis_index("s")
        wid = s * 2 + c
        rows_per = _NPHARM // 16
        pltpu.sync_copy(zeros_hbm.at[pl.ds(s * rows_per, rows_per)],
                        acc.at[pl.ds(s * rows_per, rows_per)])
        plsc.subcore_barrier()
        base = wid * per
        si = (si0, si1)
        sr = (sr0, sr1)
        hi = pltpu.async_copy(d_hbm.at[pl.ds(base, _SCT)], idx_v.at[0], si[0])
        hr = pltpu.async_copy(y_hbm.at[pl.ds(base, _SCT)], rows_v.at[0], sr[0])
        for j in range(nb):
            hi.wait()
            hr.wait()
            if j + 1 < nb:
                off = base + (j + 1) * _SCT
                hi = pltpu.async_copy(d_hbm.at[pl.ds(off, _SCT)],
                                      idx_v.at[j + 1], si[(j + 1) % 2])
                hr = pltpu.async_copy(y_hbm.at[pl.ds(off, _SCT)],
                                      rows_v.at[(j + 1) % 2], sr[(j + 1) % 2])
            pltpu.sync_copy(rows_v.at[j % 2], acc.at[idx_v.at[j]], add=True)
        plsc.subcore_barrier()
        pltpu.sync_copy(acc.at[pl.ds(s * rows_per, rows_per)],
                        out_hbm.at[c, pl.ds(s * rows_per, rows_per)])

    return _sc_scatter


def _fin_body(a1_ref, a2_ref, hp_ref, w1_ref, w2_ref, o_ref):
    new = (a1_ref[0] + a1_ref[1]) + (a2_ref[0] + a2_ref[1]) + hp_ref[...]
    o_ref[...] = jnp.maximum(new @ w1_ref[...], 0.0) @ w2_ref[...]


def _fin_call(acc1, acc2, h_pharm, w1, w2):
    return pl.pallas_call(
        _fin_body,
        out_shape=jax.ShapeDtypeStruct((_NPHARM, 1), jnp.float32),
    )(acc1, acc2, h_pharm, w1, w2)


def kernel(x_prot, x_pharm, pp_src, pp_dst, ppp_edge_index, edge_vec_pp,
           edge_vec_ppp, prot_emb_table, prot_lin_W, prot_lin_b,
           pharm_W1, pharm_b1, pharm_W2, pharm_b2,
           epp_W1, epp_b1, epp_W2, epp_b2,
           epr_W1, epr_b1, epr_W2, epr_b2,
           cpp_W1, cpp_b1, cpp_W2, cpp_b2, cpp_P,
           crev_W1, crev_b1, crev_W2, crev_b2, crev_P,
           cppp_W1, cppp_b1, cppp_W2, cppp_b2, cppp_P,
           fin_W1, fin_W2):
    row = lambda v: v.reshape(1, -1)
    h_pharm = _pharm_call(x_pharm, pharm_W1, row(pharm_b1),
                          pharm_W2, row(pharm_b2))
    h_prot = _prot_call(x_prot.astype(jnp.int32).reshape(_NPROT, 1),
                        prot_emb_table, prot_lin_W, row(prot_lin_b))

    src1 = pp_src.astype(jnp.int32)
    dst1 = pp_dst.astype(jnp.int32)
    src2 = ppp_edge_index[0].astype(jnp.int32)
    dst2 = ppp_edge_index[1].astype(jnp.int32)
    gath = _sc_gather_kernel()(h_prot.astype(jnp.bfloat16),
                               h_pharm.astype(jnp.bfloat16),
                               src1, dst1, src2, dst2)

    rmat = jnp.asarray(_R_NP)
    csmat = jnp.asarray(_CS_NP)
    esmat = jnp.asarray(_ES_NP)
    offmat = jnp.asarray(np.broadcast_to(
        np.linspace(0.0, 6.0, 50, dtype=np.float32).reshape(50, 1),
        (50, _EB)).copy())
    col = lambda v: v.reshape(-1, 1)
    y1 = _edge_conv(gath, edge_vec_pp.T, 0, _E1 // _EB, _E1,
                    epr_W1, col(epr_b1), epr_W2, row(epr_b2),
                    cpp_W1, row(cpp_b1), cpp_W2, row(cpp_b2),
                    rmat, csmat, esmat, offmat, cpp_P[:, :_NS])
    y2 = _edge_conv(gath, edge_vec_ppp.T, 2 * _E1 // _EB,
                    2 * _E1 // _EB + _E2 // _EB, _E2,
                    epp_W1, col(epp_b1), epp_W2, row(epp_b2),
                    cppp_W1, row(cppp_b1), cppp_W2, row(cppp_b2),
                    rmat, csmat, esmat, offmat, cppp_P[:, :_NS])

    zeros = jnp.zeros((_NPHARM, _NS), jnp.float32)
    acc1 = _sc_scatter_kernel(_E1)(y1, dst1, zeros)
    acc2 = _sc_scatter_kernel(_E2)(y2, dst2, zeros)
    return _fin_call(acc1, acc2, h_pharm, fin_W1, fin_W2)


# speedup vs baseline: 1.0704x; 1.0704x over previous
import functools

import numpy as np
import jax
import jax.numpy as jnp
from jax import lax
from jax.experimental import pallas as pl
from jax.experimental.pallas import tpu as pltpu
from jax.experimental.pallas import tpu_sc as plsc

_NS = 32
_SH = 9
_WN = _NS * _SH
_NCLS = 14
_NPROT = 50000
_NPHARM = 10000
_NTAB = _NPROT + _NPHARM
_E1 = 320000
_E2 = 128000
_NE_ALL = 2 * _E1 + 2 * _E2

_EB = 2560
_NW = 32
_GC = 1000
_SCC = 80

_R_NP = np.zeros((_NS, _WN), np.float32)
for _i in range(_NS):
    _R_NP[_i, _i * _SH:(_i + 1) * _SH] = 1.0
_S_NP = np.zeros((_SH, _WN), np.float32)
for _j in range(_SH):
    _S_NP[_j, _j::_SH] = 1.0

_S3, _S5, _S15 = 3.0 ** 0.5, 5.0 ** 0.5, 15.0 ** 0.5
_C_NP = np.zeros((9, 9), np.float32)
_C_NP[1, 1] = _S3
_C_NP[2, 2] = _S3
_C_NP[0, 3] = _S3
_C_NP[6, 4] = _S15
_C_NP[7, 5] = _S15
_C_NP[5, 6] = 1.5 * _S5
_C_NP[8, 7] = _S15
_C_NP[3, 8] = 0.5 * _S15
_C_NP[4, 8] = -0.5 * _S15
_E_NP = np.zeros((9,), np.float32)
_E_NP[0] = 1.0
_E_NP[6] = -0.5 * _S5
_CS_NP = _C_NP @ _S_NP
_ES_NP = (_E_NP @ _S_NP).reshape(1, _WN)


def _pharm_body(x_ref, w1_ref, b1_ref, w2_ref, b2_ref, o_ref):
    h = jnp.maximum(x_ref[...] @ w1_ref[...] + b1_ref[...], 0.0)
    o_ref[...] = h @ w2_ref[...] + b2_ref[...]


def _pharm_call(x, w1, b1, w2, b2):
    return pl.pallas_call(
        _pharm_body,
        out_shape=jax.ShapeDtypeStruct((_NPHARM, _NS), jnp.float32),
    )(x, w1, b1, w2, b2)


def _prot_body(idx_ref, tab_ref, w_ref, b_ref, o_ref):
    t2 = jnp.maximum(tab_ref[...], 0.0) @ w_ref[...] + b_ref[...]
    idx = idx_ref[...]
    onehot = (idx == lax.broadcasted_iota(jnp.int32, (idx.shape[0], _NCLS), 1)
              ).astype(jnp.float32)
    o_ref[...] = onehot @ t2


def _prot_call(idx2d, tab, w, b):
    blk = 10000
    return pl.pallas_call(
        _prot_body,
        grid=(_NPROT // blk,),
        in_specs=[
            pl.BlockSpec((blk, 1), lambda i: (i, 0)),
            pl.BlockSpec((_NCLS, _NS), lambda i: (0, 0)),
            pl.BlockSpec((_NS, _NS), lambda i: (0, 0)),
            pl.BlockSpec((1, _NS), lambda i: (0, 0)),
        ],
        out_specs=pl.BlockSpec((blk, _NS), lambda i: (i, 0)),
        out_shape=jax.ShapeDtypeStruct((_NPROT, _NS), jnp.float32),
    )(idx2d, tab, w, b)


@functools.lru_cache(maxsize=None)
def _sc_mesh():
    return plsc.VectorSubcoreMesh(core_axis_name="c", subcore_axis_name="s")


@functools.lru_cache(maxsize=None)
def _sc_gather_kernel():
    @functools.partial(
        pl.kernel,
        out_type=jax.ShapeDtypeStruct((_NE_ALL, _NS), jnp.float32),
        mesh=_sc_mesh(),
        scratch_types=[
            pltpu.VMEM((_E1 // _NW,), jnp.int32),
            pltpu.VMEM((2, _GC, _NS), jnp.float32),
            pltpu.SemaphoreType.DMA,
            pltpu.SemaphoreType.DMA,
            pltpu.SemaphoreType.DMA,
            pltpu.SemaphoreType.DMA,
        ],
        compiler_params=pltpu.CompilerParams(use_tc_tiling_on_sc=False),
    )
    def _sc_gather(prot_hbm, pharm_hbm, i1_hbm, i2_hbm, i3_hbm, i4_hbm,
                   out_hbm, idx_v, rows_v, sg0, sg1, sw0, sw1):
        wid = lax.axis_index("s") * 2 + lax.axis_index("c")
        sg = (sg0, sg1)
        sw = (sw0, sw1)

        def seg(tab_hbm, idx_hbm, n, out_base):
            per = n // _NW
            nb = per // _GC
            base = wid * per
            pltpu.sync_copy(idx_hbm.at[pl.ds(base, per)],
                            idx_v.at[pl.ds(0, per)])
            g = pltpu.async_copy(tab_hbm.at[idx_v.at[pl.ds(0, _GC)]],
                                 rows_v.at[0], sg[0])
            prev_w = None
            for j in range(nb):
                g.wait()
                if prev_w is not None:
                    prev_w.wait()
                if j + 1 < nb:
                    g = pltpu.async_copy(
                        tab_hbm.at[idx_v.at[pl.ds((j + 1) * _GC, _GC)]],
                        rows_v.at[(j + 1) % 2], sg[(j + 1) % 2])
                prev_w = pltpu.async_copy(
                    rows_v.at[j % 2],
                    out_hbm.at[pl.ds(out_base + base + j * _GC, _GC)],
                    sw[j % 2])
            prev_w.wait()

        seg(prot_hbm, i1_hbm, _E1, 0)
        seg(pharm_hbm, i2_hbm, _E1, _E1)
        seg(pharm_hbm, i3_hbm, _E2, 2 * _E1)
        seg(pharm_hbm, i4_hbm, _E2, 2 * _E1 + _E2)

    return _sc_gather


def _edge_body(vt_ref, xs_ref, xd_ref, dw1_ref, db1_ref, dw2_ref, db2_ref,
               cw1_ref, cb1_ref, cw2_ref, cb2_ref, r_ref, cs_ref, es_ref,
               off_ref, p_ref, o_ref, *, B):
    bf = jnp.bfloat16
    dot = functools.partial(jax.lax.dot, preferred_element_type=jnp.float32)
    dot0 = lambda a, b: lax.dot_general(
        a, b, (((0,), (0,)), ((), ())), preferred_element_type=jnp.float32)

    vt = vt_ref[...]
    ss = jnp.sum(vt * vt, axis=0, keepdims=True)
    rinv = lax.rsqrt(ss + 1e-12)
    dist = ss * rinv
    u = vt * rinv
    ur = jnp.concatenate([u[1:3], u[0:1]], axis=0)
    mono = jnp.concatenate([u, u * u, u * ur], axis=0)
    sh_tile = dot0(mono.astype(bf), cs_ref[...].astype(bf)) + es_ref[...]

    step = 6.0 / 49.0
    dd = dist - off_ref[...]
    g = jnp.exp((-0.5 / (step * step)) * dd * dd)
    hd = jnp.maximum(dot0(dw1_ref[...], g) + db1_ref[...], 0.0)

    cw1 = cw1_ref[...]
    dcw = dot(dw2_ref[...], cw1[0:_NS])
    bfold = cb1_ref[...] + dot(db2_ref[...], cw1[0:_NS])
    xs = xs_ref[...].astype(bf)
    hid = jnp.maximum(
        dot0(hd.astype(bf), dcw.astype(bf))
        + dot(xs, cw1[_NS:2 * _NS].astype(bf))
        + dot(xd_ref[...].astype(bf), cw1[2 * _NS:].astype(bf))
        + bfold, 0.0)
    w = dot(hid.astype(bf), cw2_ref[...].astype(bf)) + cb2_ref[...]
    m = dot(xs, r_ref[...].astype(bf)) * sh_tile * w
    o_ref[...] = dot(m.astype(bf), p_ref[...].astype(bf))


def _edge_conv(gath, vec_t, xs_off, xd_off, n_edges,
               dw1, db1, dw2, db2, cw1, cb1, cw2, cb2, rmat, csmat, esmat,
               offmat, p32):
    B = _EB
    nb = n_edges // B
    full = lambda shape: pl.BlockSpec(shape, lambda i: (0, 0))
    return pl.pallas_call(
        functools.partial(_edge_body, B=B),
        grid=(nb,),
        in_specs=[
            pl.BlockSpec((3, B), lambda i: (0, i)),
            pl.BlockSpec((B, _NS), lambda i, o=xs_off: (i + o, 0)),
            pl.BlockSpec((B, _NS), lambda i, o=xd_off: (i + o, 0)),
            full((50, _NS)), full((_NS, 1)), full((_NS, _NS)), full((1, _NS)),
            full((3 * _NS, 3 * _NS)), full((1, 3 * _NS)),
            full((3 * _NS, _WN)), full((1, _WN)),
            full((_NS, _WN)), full((_SH, _WN)), full((1, _WN)),
            full((50, B)),
            full((_WN, _NS)),
        ],
        out_specs=pl.BlockSpec((B, _NS), lambda i: (i, 0)),
        out_shape=jax.ShapeDtypeStruct((n_edges, _NS), jnp.float32),
    )(vec_t, gath, gath, dw1, db1, dw2, db2, cw1, cb1, cw2, cb2,
      rmat, csmat, esmat, offmat, p32)


_SCT = 1000


@functools.lru_cache(maxsize=None)
def _sc_scatter_kernel(n_edges):
    per = n_edges // _NW
    nb = per // _SCT

    @functools.partial(
        pl.kernel,
        out_type=jax.ShapeDtypeStruct((2, _NPHARM, _NS), jnp.float32),
        mesh=_sc_mesh(),
        scratch_types=[
            pltpu.VMEM_SHARED((_NPHARM, _NS), jnp.float32),
            pltpu.VMEM((nb, _SCT), jnp.int32),
            pltpu.VMEM((2, _SCT, _NS), jnp.float32),
            pltpu.SemaphoreType.DMA,
            pltpu.SemaphoreType.DMA,
            pltpu.SemaphoreType.DMA,
            pltpu.SemaphoreType.DMA,
        ],
        compiler_params=pltpu.CompilerParams(use_tc_tiling_on_sc=False),
    )
    def _sc_scatter(y_hbm, d_hbm, zeros_hbm, out_hbm, acc, idx_v, rows_v,
                    si0, si1, sr0, sr1):
        c = lax.axis_index("c")
        s = lax.axis_index("s")
        wid = s * 2 + c
        rows_per = _NPHARM // 16
        pltpu.sync_copy(zeros_hbm.at[pl.ds(s * rows_per, rows_per)],
                        acc.at[pl.ds(s * rows_per, rows_per)])
        plsc.subcore_barrier()
        base = wid * per
        si = (si0, si1)
        sr = (sr0, sr1)
        hi = pltpu.async_copy(d_hbm.at[pl.ds(base, _SCT)], idx_v.at[0], si[0])
        hr = pltpu.async_copy(y_hbm.at[pl.ds(base, _SCT)], rows_v.at[0], sr[0])
        for j in range(nb):
            hi.wait()
            hr.wait()
            if j + 1 < nb:
                off = base + (j + 1) * _SCT
                hi = pltpu.async_copy(d_hbm.at[pl.ds(off, _SCT)],
                                      idx_v.at[j + 1], si[(j + 1) % 2])
                hr = pltpu.async_copy(y_hbm.at[pl.ds(off, _SCT)],
                                      rows_v.at[(j + 1) % 2], sr[(j + 1) % 2])
            pltpu.sync_copy(rows_v.at[j % 2], acc.at[idx_v.at[j]], add=True)
        plsc.subcore_barrier()
        pltpu.sync_copy(acc.at[pl.ds(s * rows_per, rows_per)],
                        out_hbm.at[c, pl.ds(s * rows_per, rows_per)])

    return _sc_scatter


def _fin_body(a1_ref, a2_ref, hp_ref, w1_ref, w2_ref, o_ref):
    new = (a1_ref[0] + a1_ref[1]) + (a2_ref[0] + a2_ref[1]) + hp_ref[...]
    o_ref[...] = jnp.maximum(new @ w1_ref[...], 0.0) @ w2_ref[...]


def _fin_call(acc1, acc2, h_pharm, w1, w2):
    return pl.pallas_call(
        _fin_body,
        out_shape=jax.ShapeDtypeStruct((_NPHARM, 1), jnp.float32),
    )(acc1, acc2, h_pharm, w1, w2)


def kernel(x_prot, x_pharm, pp_src, pp_dst, ppp_edge_index, edge_vec_pp,
           edge_vec_ppp, prot_emb_table, prot_lin_W, prot_lin_b,
           pharm_W1, pharm_b1, pharm_W2, pharm_b2,
           epp_W1, epp_b1, epp_W2, epp_b2,
           epr_W1, epr_b1, epr_W2, epr_b2,
           cpp_W1, cpp_b1, cpp_W2, cpp_b2, cpp_P,
           crev_W1, crev_b1, crev_W2, crev_b2, crev_P,
           cppp_W1, cppp_b1, cppp_W2, cppp_b2, cppp_P,
           fin_W1, fin_W2):
    row = lambda v: v.reshape(1, -1)
    h_pharm = _pharm_call(x_pharm, pharm_W1, row(pharm_b1),
                          pharm_W2, row(pharm_b2))
    h_prot = _prot_call(x_prot.astype(jnp.int32).reshape(_NPROT, 1),
                        prot_emb_table, prot_lin_W, row(prot_lin_b))

    src1 = pp_src.astype(jnp.int32)
    dst1 = pp_dst.astype(jnp.int32)
    src2 = ppp_edge_index[0].astype(jnp.int32)
    dst2 = ppp_edge_index[1].astype(jnp.int32)
    gath = _sc_gather_kernel()(h_prot, h_pharm, src1, dst1, src2, dst2)

    rmat = jnp.asarray(_R_NP)
    csmat = jnp.asarray(_CS_NP)
    esmat = jnp.asarray(_ES_NP)
    offmat = jnp.asarray(np.broadcast_to(
        np.linspace(0.0, 6.0, 50, dtype=np.float32).reshape(50, 1),
        (50, _EB)).copy())
    col = lambda v: v.reshape(-1, 1)
    y1 = _edge_conv(gath, edge_vec_pp.T, 0, _E1 // _EB, _E1,
                    epr_W1, col(epr_b1), epr_W2, row(epr_b2),
                    cpp_W1, row(cpp_b1), cpp_W2, row(cpp_b2),
                    rmat, csmat, esmat, offmat, cpp_P[:, :_NS])
    y2 = _edge_conv(gath, edge_vec_ppp.T, 2 * _E1 // _EB,
                    2 * _E1 // _EB + _E2 // _EB, _E2,
                    epp_W1, col(epp_b1), epp_W2, row(epp_b2),
                    cppp_W1, row(cppp_b1), cppp_W2, row(cppp_b2),
                    rmat, csmat, esmat, offmat, cppp_P[:, :_NS])

    zeros = jnp.zeros((_NPHARM, _NS), jnp.float32)
    acc1 = _sc_scatter_kernel(_E1)(y1, dst1, zeros)
    acc2 = _sc_scatter_kernel(_E2)(y2, dst2, zeros)
    return _fin_call(acc1, acc2, h_pharm, fin_W1, fin_W2)


# speedup vs baseline: 1.0719x; 1.0015x over previous
import functools

import numpy as np
import jax
import jax.numpy as jnp
from jax import lax
from jax.experimental import pallas as pl
from jax.experimental.pallas import tpu as pltpu
from jax.experimental.pallas import tpu_sc as plsc

_NS = 32
_SH = 9
_WN = _NS * _SH
_NCLS = 14
_NPROT = 50000
_NPHARM = 10000
_NTAB = _NPROT + _NPHARM
_E1 = 320000
_E2 = 128000
_NE_ALL = 2 * _E1 + 2 * _E2

_EB = 2560
_NW = 32
_GC = 1000
_SCC = 80

_R_NP = np.zeros((_NS, _WN), np.float32)
for _i in range(_NS):
    _R_NP[_i, _i * _SH:(_i + 1) * _SH] = 1.0
_S_NP = np.zeros((_SH, _WN), np.float32)
for _j in range(_SH):
    _S_NP[_j, _j::_SH] = 1.0

_S3, _S5, _S15 = 3.0 ** 0.5, 5.0 ** 0.5, 15.0 ** 0.5
_C_NP = np.zeros((9, 9), np.float32)
_C_NP[1, 1] = _S3
_C_NP[2, 2] = _S3
_C_NP[0, 3] = _S3
_C_NP[6, 4] = _S15
_C_NP[7, 5] = _S15
_C_NP[5, 6] = 1.5 * _S5
_C_NP[8, 7] = _S15
_C_NP[3, 8] = 0.5 * _S15
_C_NP[4, 8] = -0.5 * _S15
_E_NP = np.zeros((9,), np.float32)
_E_NP[0] = 1.0
_E_NP[6] = -0.5 * _S5
_CS_NP = _C_NP @ _S_NP
_ES_NP = (_E_NP @ _S_NP).reshape(1, _WN)


def _pharm_body(x_ref, w1_ref, b1_ref, w2_ref, b2_ref, o_ref):
    h = jnp.maximum(x_ref[...] @ w1_ref[...] + b1_ref[...], 0.0)
    o_ref[...] = h @ w2_ref[...] + b2_ref[...]


def _pharm_call(x, w1, b1, w2, b2):
    return pl.pallas_call(
        _pharm_body,
        out_shape=jax.ShapeDtypeStruct((_NPHARM, _NS), jnp.float32),
    )(x, w1, b1, w2, b2)


def _prot_body(idx_ref, tab_ref, w_ref, b_ref, o_ref):
    t2 = jnp.maximum(tab_ref[...], 0.0) @ w_ref[...] + b_ref[...]
    idx = idx_ref[...]
    onehot = (idx == lax.broadcasted_iota(jnp.int32, (idx.shape[0], _NCLS), 1)
              ).astype(jnp.float32)
    o_ref[...] = onehot @ t2


def _prot_call(idx2d, tab, w, b):
    blk = 10000
    return pl.pallas_call(
        _prot_body,
        grid=(_NPROT // blk,),
        in_specs=[
            pl.BlockSpec((blk, 1), lambda i: (i, 0)),
            pl.BlockSpec((_NCLS, _NS), lambda i: (0, 0)),
            pl.BlockSpec((_NS, _NS), lambda i: (0, 0)),
            pl.BlockSpec((1, _NS), lambda i: (0, 0)),
        ],
        out_specs=pl.BlockSpec((blk, _NS), lambda i: (i, 0)),
        out_shape=jax.ShapeDtypeStruct((_NPROT, _NS), jnp.float32),
    )(idx2d, tab, w, b)


@functools.lru_cache(maxsize=None)
def _sc_mesh():
    return plsc.VectorSubcoreMesh(core_axis_name="c", subcore_axis_name="s")


@functools.lru_cache(maxsize=None)
def _sc_gather_kernel():
    @functools.partial(
        pl.kernel,
        out_type=jax.ShapeDtypeStruct((_NE_ALL, _NS), jnp.float32),
        mesh=_sc_mesh(),
        scratch_types=[
            pltpu.VMEM((_E1 // _NW,), jnp.int32),
            pltpu.VMEM((2, _GC, _NS), jnp.float32),
            pltpu.SemaphoreType.DMA,
            pltpu.SemaphoreType.DMA,
            pltpu.SemaphoreType.DMA,
            pltpu.SemaphoreType.DMA,
        ],
        compiler_params=pltpu.CompilerParams(use_tc_tiling_on_sc=False),
    )
    def _sc_gather(prot_hbm, pharm_hbm, i1_hbm, i2_hbm, i3_hbm, i4_hbm,
                   out_hbm, idx_v, rows_v, sg0, sg1, sw0, sw1):
        wid = lax.axis_index("s") * 2 + lax.axis_index("c")
        sg = (sg0, sg1)
        sw = (sw0, sw1)

        def seg(tab_hbm, idx_hbm, n, out_base):
            per = n // _NW
            nb = per // _GC
            base = wid * per
            pltpu.sync_copy(idx_hbm.at[pl.ds(base, per)],
                            idx_v.at[pl.ds(0, per)])
            g = pltpu.async_copy(tab_hbm.at[idx_v.at[pl.ds(0, _GC)]],
                                 rows_v.at[0], sg[0])
            prev_w = None
            for j in range(nb):
                g.wait()
                if prev_w is not None:
                    prev_w.wait()
                if j + 1 < nb:
                    g = pltpu.async_copy(
                        tab_hbm.at[idx_v.at[pl.ds((j + 1) * _GC, _GC)]],
                        rows_v.at[(j + 1) % 2], sg[(j + 1) % 2])
                prev_w = pltpu.async_copy(
                    rows_v.at[j % 2],
                    out_hbm.at[pl.ds(out_base + base + j * _GC, _GC),
                               pl.ds(0, _NS)],
                    sw[j % 2])
            prev_w.wait()

        seg(prot_hbm, i1_hbm, _E1, 0)
        seg(pharm_hbm, i2_hbm, _E1, _E1)
        seg(pharm_hbm, i3_hbm, _E2, 2 * _E1)
        seg(pharm_hbm, i4_hbm, _E2, 2 * _E1 + _E2)

    return _sc_gather


def _edge_body(vt_ref, xs_ref, xd_ref, dw1_ref, db1_ref, dw2_ref, db2_ref,
               cw1_ref, cb1_ref, cw2_ref, cb2_ref, r_ref, cs_ref, es_ref,
               off_ref, p_ref, o_ref, *, B):
    bf = jnp.bfloat16
    dot = functools.partial(jax.lax.dot, preferred_element_type=jnp.float32)
    dot0 = lambda a, b: lax.dot_general(
        a, b, (((0,), (0,)), ((), ())), preferred_element_type=jnp.float32)

    vt = vt_ref[...]
    ss = jnp.sum(vt * vt, axis=0, keepdims=True)
    rinv = lax.rsqrt(ss + 1e-12)
    dist = ss * rinv
    u = vt * rinv
    ur = jnp.concatenate([u[1:3], u[0:1]], axis=0)
    mono = jnp.concatenate([u, u * u, u * ur], axis=0)
    sh_tile = dot0(mono.astype(bf), cs_ref[...].astype(bf)) + es_ref[...]

    step = 6.0 / 49.0
    dd = dist - off_ref[...]
    g = jnp.exp((-0.5 / (step * step)) * dd * dd)
    hd = jnp.maximum(dot0(dw1_ref[...], g) + db1_ref[...], 0.0)

    cw1 = cw1_ref[...]
    dcw = dot(dw2_ref[...], cw1[0:_NS])
    bfold = cb1_ref[...] + dot(db2_ref[...], cw1[0:_NS])
    xs = xs_ref[...].astype(bf)
    hid = jnp.maximum(
        dot0(hd.astype(bf), dcw.astype(bf))
        + dot(xs, cw1[_NS:2 * _NS].astype(bf))
        + dot(xd_ref[...].astype(bf), cw1[2 * _NS:].astype(bf))
        + bfold, 0.0)
    w = dot(hid.astype(bf), cw2_ref[...].astype(bf)) + cb2_ref[...]
    m = dot(xs, r_ref[...].astype(bf)) * sh_tile * w
    o_ref[...] = dot(m.astype(bf), p_ref[...].astype(bf))


def _edge_conv(gath, vec_t, xs_off, xd_off, n_edges,
               dw1, db1, dw2, db2, cw1, cb1, cw2, cb2, rmat, csmat, esmat,
               offmat, p32):
    B = _EB
    nb = n_edges // B
    full = lambda shape: pl.BlockSpec(shape, lambda i: (0, 0))
    return pl.pallas_call(
        functools.partial(_edge_body, B=B),
        grid=(nb,),
        in_specs=[
            pl.BlockSpec((3, B), lambda i: (0, i)),
            pl.BlockSpec((B, _NS), lambda i, o=xs_off: (i + o, 0)),
            pl.BlockSpec((B, _NS), lambda i, o=xd_off: (i + o, 0)),
            full((50, _NS)), full((_NS, 1)), full((_NS, _NS)), full((1, _NS)),
            full((3 * _NS, 3 * _NS)), full((1, 3 * _NS)),
            full((3 * _NS, _WN)), full((1, _WN)),
            full((_NS, _WN)), full((_SH, _WN)), full((1, _WN)),
            full((50, B)),
            full((_WN, _NS)),
        ],
        out_specs=pl.BlockSpec((B, _NS), lambda i: (i, 0)),
        out_shape=jax.ShapeDtypeStruct((n_edges, _NS), jnp.float32),
    )(vec_t, gath, gath, dw1, db1, dw2, db2, cw1, cb1, cw2, cb2,
      rmat, csmat, esmat, offmat, p32)


_SCT = 1000


@functools.lru_cache(maxsize=None)
def _sc_scatter_kernel(n_edges):
    per = n_edges // _NW
    nb = per // _SCT

    @functools.partial(
        pl.kernel,
        out_type=jax.ShapeDtypeStruct((2, _NPHARM, _NS), jnp.float32),
        mesh=_sc_mesh(),
        scratch_types=[
            pltpu.VMEM_SHARED((_NPHARM, _NS), jnp.float32),
            pltpu.VMEM((nb, _SCT), jnp.int32),
            pltpu.VMEM((2, _SCT, _NS), jnp.float32),
            pltpu.SemaphoreType.DMA,
            pltpu.SemaphoreType.DMA,
            pltpu.SemaphoreType.DMA,
            pltpu.SemaphoreType.DMA,
        ],
        compiler_params=pltpu.CompilerParams(use_tc_tiling_on_sc=False),
    )
    def _sc_scatter(y_hbm, d_hbm, zeros_hbm, out_hbm, acc, idx_v, rows_v,
                    si0, si1, sr0, sr1):
        c = lax.axis_index("c")
        s = lax.axis_index("s")
        wid = s * 2 + c
        rows_per = _NPHARM // 16
        pltpu.sync_copy(zeros_hbm.at[pl.ds(s * rows_per, rows_per)],
                        acc.at[pl.ds(s * rows_per, rows_per)])
        plsc.subcore_barrier()
        base = wid * per
        si = (si0, si1)
        sr = (sr0, sr1)
        hi = pltpu.async_copy(d_hbm.at[pl.ds(base, _SCT)], idx_v.at[0], si[0])
        hr = pltpu.async_copy(y_hbm.at[pl.ds(base, _SCT), pl.ds(0, _NS)],
                              rows_v.at[0], sr[0])
        for j in range(nb):
            hi.wait()
            hr.wait()
            if j + 1 < nb:
                off = base + (j + 1) * _SCT
                hi = pltpu.async_copy(d_hbm.at[pl.ds(off, _SCT)],
                                      idx_v.at[j + 1], si[(j + 1) % 2])
                hr = pltpu.async_copy(y_hbm.at[pl.ds(off, _SCT),
                                               pl.ds(0, _NS)],
                                      rows_v.at[(j + 1) % 2], sr[(j + 1) % 2])
            pltpu.sync_copy(rows_v.at[j % 2], acc.at[idx_v.at[j]], add=True)
        plsc.subcore_barrier()
        pltpu.sync_copy(acc.at[pl.ds(s * rows_per, rows_per)],
                        out_hbm.at[c, pl.ds(s * rows_per, rows_per)])

    return _sc_scatter


def _fin_body(a1_ref, a2_ref, hp_ref, w1_ref, w2_ref, o_ref):
    new = (a1_ref[0] + a1_ref[1]) + (a2_ref[0] + a2_ref[1]) + hp_ref[...]
    o_ref[...] = jnp.maximum(new @ w1_ref[...], 0.0) @ w2_ref[...]


def _fin_call(acc1, acc2, h_pharm, w1, w2):
    return pl.pallas_call(
        _fin_body,
        out_shape=jax.ShapeDtypeStruct((_NPHARM, 1), jnp.float32),
    )(acc1, acc2, h_pharm, w1, w2)


def kernel(x_prot, x_pharm, pp_src, pp_dst, ppp_edge_index, edge_vec_pp,
           edge_vec_ppp, prot_emb_table, prot_lin_W, prot_lin_b,
           pharm_W1, pharm_b1, pharm_W2, pharm_b2,
           epp_W1, epp_b1, epp_W2, epp_b2,
           epr_W1, epr_b1, epr_W2, epr_b2,
           cpp_W1, cpp_b1, cpp_W2, cpp_b2, cpp_P,
           crev_W1, crev_b1, crev_W2, crev_b2, crev_P,
           cppp_W1, cppp_b1, cppp_W2, cppp_b2, cppp_P,
           fin_W1, fin_W2):
    row = lambda v: v.reshape(1, -1)
    h_pharm = _pharm_call(x_pharm, pharm_W1, row(pharm_b1),
                          pharm_W2, row(pharm_b2))
    h_prot = _prot_call(x_prot.astype(jnp.int32).reshape(_NPROT, 1),
                        prot_emb_table, prot_lin_W, row(prot_lin_b))

    src1 = pp_src.astype(jnp.int32)
    dst1 = pp_dst.astype(jnp.int32)
    src2 = ppp_edge_index[0].astype(jnp.int32)
    dst2 = ppp_edge_index[1].astype(jnp.int32)
    gath = _sc_gather_kernel()(h_prot, h_pharm, src1, dst1, src2, dst2)

    rmat = jnp.asarray(_R_NP)
    csmat = jnp.asarray(_CS_NP)
    esmat = jnp.asarray(_ES_NP)
    offmat = jnp.asarray(np.broadcast_to(
        np.linspace(0.0, 6.0, 50, dtype=np.float32).reshape(50, 1),
        (50, _EB)).copy())
    col = lambda v: v.reshape(-1, 1)
    y1 = _edge_conv(gath, edge_vec_pp.T, 0, _E1 // _EB, _E1,
                    epr_W1, col(epr_b1), epr_W2, row(epr_b2),
                    cpp_W1, row(cpp_b1), cpp_W2, row(cpp_b2),
                    rmat, csmat, esmat, offmat, cpp_P[:, :_NS])
    y2 = _edge_conv(gath, edge_vec_ppp.T, 2 * _E1 // _EB,
                    2 * _E1 // _EB + _E2 // _EB, _E2,
                    epp_W1, col(epp_b1), epp_W2, row(epp_b2),
                    cppp_W1, row(cppp_b1), cppp_W2, row(cppp_b2),
                    rmat, csmat, esmat, offmat, cppp_P[:, :_NS])

    zeros = jnp.zeros((_NPHARM, _NS), jnp.float32)
    acc1 = _sc_scatter_kernel(_E1)(y1, dst1, zeros)
    acc2 = _sc_scatter_kernel(_E2)(y2, dst2, zeros)
    return _fin_call(acc1, acc2, h_pharm, fin_W1, fin_W2)
